# baseline (speedup 1.0000x reference)
import functools
import jax
import jax.numpy as jnp
from jax import lax
from jax.experimental import pallas as pl
from jax.experimental.pallas import tpu as pltpu
from jax.experimental.pallas import tpu_sc as plsc

N = 20000
LANES = 128
ROWS = 160
NPAD = ROWS * LANES
MAX_OUT = 200
SEL_PAD = 256
NMS_THRESH = 0.4
V0 = 0.1
V1 = 0.2
NEG_INF = float("-inf")
INT_MAX = 2**31 - 1


def _nms_body(x_ref, out_ref, sel_ref, chans_ref, s_ref):
    f32 = jnp.float32
    sc = x_ref[0]
    dx = x_ref[1] * f32(V0)
    dy = x_ref[2] * f32(V0)
    dw = x_ref[3] * f32(V1)
    dh = x_ref[4] * f32(V1)
    x_a = x_ref[5]
    y_a = x_ref[6]
    w_a = x_ref[7]
    h_a = x_ref[8]
    xc = dx * w_a + x_a
    yc = dy * h_a + y_a
    w = jnp.exp(dw) * w_a
    h = jnp.exp(dh) * h_a
    ymin = yc - h / 2
    xmin = xc - w / 2
    ymax = yc + h / 2
    xmax = xc + w / 2
    chans_ref[0] = ymin
    chans_ref[1] = xmin
    chans_ref[2] = ymax
    chans_ref[3] = xmax
    chans_ref[4] = (ymax - ymin) * (xmax - xmin)
    s_ref[...] = jnp.where(sc >= f32(NMS_THRESH), sc, NEG_INF)
    sel_ref[...] = jnp.full((SEL_PAD, 1), NPAD - 1, jnp.int32)

    gid = (lax.broadcasted_iota(jnp.int32, (ROWS, LANES), 0) * LANES
           + lax.broadcasted_iota(jnp.int32, (ROWS, LANES), 1))
    lane_iota = lax.broadcasted_iota(jnp.int32, (1, LANES), 1)
    out_iota = lax.broadcasted_iota(jnp.int32, (1, 16), 1)

    def body(i, carry):
        s = s_ref[...]
        m = jnp.max(s)
        ok = m > NEG_INF
        idx = jnp.min(jnp.where(s == m, gid, INT_MAX))
        r = idx // LANES
        lane = idx - r * LANES
        lonehot = lane_iota == lane
        vals = []
        for c in range(4):
            rv = chans_ref[c, pl.ds(r, 1), :]
            vals.append(jnp.sum(jnp.where(lonehot, rv, f32(0.0))))
        sy0, sx0, sy1, sx1 = vals
        area1 = (sy1 - sy0) * (sx1 - sx0)
        iy0 = jnp.maximum(sy0, chans_ref[0])
        ix0 = jnp.maximum(sx0, chans_ref[1])
        iy1 = jnp.minimum(sy1, chans_ref[2])
        ix1 = jnp.minimum(sx1, chans_ref[3])
        inter = (jnp.maximum(iy1 - iy0, f32(0.0))
                 * jnp.maximum(ix1 - ix0, f32(0.0)))
        iou = inter / (area1 + chans_ref[4] - inter + f32(1e-8))
        kill = (iou > f32(NMS_THRESH)) | (gid == idx)
        s_ref[...] = jnp.where(kill, NEG_INF, s)
        okf = jnp.where(ok, f32(1.0), f32(0.0))
        row = jnp.zeros((1, 16), jnp.float32)
        for c, v in enumerate(vals):
            row = jnp.where(out_iota == c, v, row)
        out_ref[pl.ds(i, 1), :] = row * okf
        idx_eff = jnp.where(ok, idx, NPAD - 1)
        sel_ref[pl.ds(i, 1), :] = jnp.zeros((1, 1), jnp.int32) + idx_eff
        return carry

    lax.fori_loop(0, MAX_OUT, body, 0)


def _make_sc_lnd():
    mesh = plsc.VectorSubcoreMesh(core_axis_name="c", subcore_axis_name="s")

    @functools.partial(
        pl.kernel, mesh=mesh,
        out_type=jax.ShapeDtypeStruct((SEL_PAD, 16), jnp.float32),
        scratch_types=[
            pltpu.VMEM((8,), jnp.int32),
            pltpu.VMEM((8, 128), jnp.float32),
            pltpu.VMEM((8, 16), jnp.float32),
            pltpu.SemaphoreType.DMA,
        ],
    )
    def sc_lnd(xrow_hbm, sel_hbm, out_hbm, idx_v, rows_v, outr_v, sem):
        f32 = jnp.float32
        wid = lax.axis_index("s") * 2 + lax.axis_index("c")
        base = wid * 8
        pltpu.sync_copy(sel_hbm.at[pl.ds(base, 8)], idx_v)
        pltpu.async_copy(xrow_hbm.at[idx_v], rows_v, sem).wait()
        lane = lax.iota(jnp.int32, 16)
        for i in range(8):
            raw = rows_v[i, pl.ds(0, 16)]
            mult = rows_v[i, pl.ds(16, 16)]
            add = rows_v[i, pl.ds(32, 16)]
            dec = (raw * f32(V0)) * mult + add
            outr_v[i] = jnp.where(lane < 10, dec, f32(0.0))
        pltpu.sync_copy(outr_v, out_hbm.at[pl.ds(base, 8)])

    return sc_lnd


def kernel(cls_pred, reg_pred, lnd_pred, anchors):
    scores = cls_pred[0, :, 1]
    x9 = jnp.concatenate([scores[:, None], reg_pred[0], anchors], axis=1)
    xt = jnp.pad(x9.T, ((0, 0), (0, NPAD - N))).reshape(9, ROWS, LANES)
    boxes16, sel = pl.pallas_call(
        _nms_body,
        out_shape=(
            jax.ShapeDtypeStruct((MAX_OUT, 16), jnp.float32),
            jax.ShapeDtypeStruct((SEL_PAD, 1), jnp.int32),
        ),
        scratch_shapes=[
            pltpu.VMEM((5, ROWS, LANES), jnp.float32),
            pltpu.VMEM((ROWS, LANES), jnp.float32),
        ],
    )(xt)
    lnd16 = jnp.pad(lnd_pred[0], ((0, 0), (0, 6)))
    mult = jnp.tile(anchors[:, 2:4], (1, 8))
    add = jnp.tile(anchors[:, 0:2], (1, 8))
    xrow = jnp.pad(jnp.concatenate([lnd16, mult, add], axis=1),
                   ((0, NPAD - N), (0, 128 - 48)))
    lnd = _make_sc_lnd()(xrow, sel[:, 0])
    return boxes16[:, :4], lnd[:MAX_OUT, :10]


# speedup vs baseline: 1.2560x; 1.2560x over previous
import jax
import jax.numpy as jnp
from jax import lax
from jax.experimental import pallas as pl
from jax.experimental.pallas import tpu as pltpu

N = 20000
LANES = 128
ROWS = 160
NPAD = ROWS * LANES
MAX_OUT = 200
NMS_THRESH = 0.4
V0 = 0.1
V1 = 0.2
NEG_INF = float("-inf")
INT_MAX = 2**31 - 1

NCH = 15


def _nms_body(x_ref, out_ref, chans_ref, s_ref, ci_ref, cc_ref):
    f32 = jnp.float32
    sc = x_ref[0]
    dx = x_ref[1] * f32(V0)
    dy = x_ref[2] * f32(V0)
    dw = x_ref[3] * f32(V1)
    dh = x_ref[4] * f32(V1)
    x_a = x_ref[15]
    y_a = x_ref[16]
    w_a = x_ref[17]
    h_a = x_ref[18]
    xc = dx * w_a + x_a
    yc = dy * h_a + y_a
    w = jnp.exp(dw) * w_a
    h = jnp.exp(dh) * h_a
    ymin = yc - h / 2
    xmin = xc - w / 2
    ymax = yc + h / 2
    xmax = xc + w / 2
    chans_ref[0] = ymin
    chans_ref[1] = xmin
    chans_ref[2] = ymax
    chans_ref[3] = xmax
    for j in range(5):
        chans_ref[4 + 2 * j] = (x_ref[5 + 2 * j] * f32(V0)) * w_a + x_a
        chans_ref[5 + 2 * j] = (x_ref[6 + 2 * j] * f32(V0)) * h_a + y_a
    chans_ref[14] = (ymax - ymin) * (xmax - xmin)
    s_ref[...] = jnp.where(sc >= f32(NMS_THRESH), sc, NEG_INF)
    ci_ref[0] = 0

    gid = (lax.broadcasted_iota(jnp.int32, (ROWS, LANES), 0) * LANES
           + lax.broadcasted_iota(jnp.int32, (ROWS, LANES), 1))
    lane_iota = lax.broadcasted_iota(jnp.int32, (1, LANES), 1)
    out_iota = lax.broadcasted_iota(jnp.int32, (1, 16), 1)

    def extract4(idx):
        r = idx // LANES
        lonehot = lane_iota == (idx - r * LANES)
        return [jnp.sum(jnp.where(lonehot, chans_ref[c, pl.ds(r, 1), :],
                                  f32(0.0))) for c in range(4)]

    def body(i, carry):
        @pl.when(ci_ref[0] == 0)
        def _slow():
            s = s_ref[...]
            m = jnp.max(s)
            idx = jnp.min(jnp.where(s == m, gid, INT_MAX))
            cs = extract4(idx)
            ci_ref[1] = idx
            for c in range(4):
                cc_ref[c] = cs[c]
            cc_ref[4] = jnp.where(m > NEG_INF, f32(1.0), f32(0.0))

        s = s_ref[...]
        idx = ci_ref[1]
        sy0 = cc_ref[0]
        sx0 = cc_ref[1]
        sy1 = cc_ref[2]
        sx1 = cc_ref[3]
        okf = cc_ref[4]
        area1 = (sy1 - sy0) * (sx1 - sx0)
        iy0 = jnp.maximum(sy0, chans_ref[0])
        ix0 = jnp.maximum(sx0, chans_ref[1])
        iy1 = jnp.minimum(sy1, chans_ref[2])
        ix1 = jnp.minimum(sx1, chans_ref[3])
        inter = (jnp.maximum(iy1 - iy0, f32(0.0))
                 * jnp.maximum(ix1 - ix0, f32(0.0)))
        iou = inter / (area1 + chans_ref[14] - inter + f32(1e-8))
        kill = (iou > f32(NMS_THRESH)) | (gid == idx)
        s_ref[...] = jnp.where(kill, NEG_INF, s)

        r = idx // LANES
        lonehot = lane_iota == (idx - r * LANES)
        row = jnp.zeros((1, 16), jnp.float32)
        for c, v in enumerate((sy0, sx0, sy1, sx1)):
            row = jnp.where(out_iota == c, v, row)
        for c in range(4, 14):
            rv = chans_ref[c, pl.ds(r, 1), :]
            v = jnp.sum(jnp.where(lonehot, rv, f32(0.0)))
            row = jnp.where(out_iota == c, v, row)
        out_ref[pl.ds(i, 1), :] = row * okf

        s2 = jnp.where(gid == idx, NEG_INF, s)
        m2 = jnp.max(s2)
        idx2 = jnp.min(jnp.where(s2 == m2, gid, INT_MAX))
        cs2 = extract4(idx2)
        killed2 = jnp.sum(jnp.where(gid == idx2,
                                    jnp.where(kill, f32(1.0), f32(0.0)),
                                    f32(0.0)))
        ci_ref[0] = jnp.where(killed2 < f32(0.5), 1, 0)
        ci_ref[1] = idx2
        for c in range(4):
            cc_ref[c] = cs2[c]
        cc_ref[4] = jnp.where(m2 > NEG_INF, f32(1.0), f32(0.0))
        return carry

    lax.fori_loop(0, MAX_OUT, body, 0)


def kernel(cls_pred, reg_pred, lnd_pred, anchors):
    scores = cls_pred[0, :, 1]
    x = jnp.concatenate(
        [scores[:, None], reg_pred[0], lnd_pred[0], anchors], axis=1)
    xt = jnp.pad(x.T, ((0, 0), (0, NPAD - N))).reshape(19, ROWS, LANES)
    out = pl.pallas_call(
        _nms_body,
        out_shape=jax.ShapeDtypeStruct((MAX_OUT, 16), jnp.float32),
        scratch_shapes=[
            pltpu.VMEM((NCH, ROWS, LANES), jnp.float32),
            pltpu.VMEM((ROWS, LANES), jnp.float32),
            pltpu.SMEM((2,), jnp.int32),
            pltpu.SMEM((8,), jnp.float32),
        ],
    )(xt)
    return out[:, :4], out[:, 4:14]


# speedup vs baseline: 1.2717x; 1.0125x over previous
import jax
import jax.numpy as jnp
from jax import lax
from jax.experimental import pallas as pl
from jax.experimental.pallas import tpu as pltpu

N = 20000
LANES = 128
ROWS = (N + LANES - 1) // LANES
ROWS = ((ROWS + 7) // 8) * 8
NPAD = ROWS * LANES
MAX_OUT = 200
NMS_THRESH = 0.4
V0 = 0.1
V1 = 0.2
NEG_INF = float("-inf")
INT_MAX = 2**31 - 1

NCH = 15


def _nms_body(x_ref, out_ref, chans_ref, s_ref):
    f32 = jnp.float32
    sc = x_ref[0]
    dx = x_ref[1] * f32(V0)
    dy = x_ref[2] * f32(V0)
    dw = x_ref[3] * f32(V1)
    dh = x_ref[4] * f32(V1)
    x_a = x_ref[15]
    y_a = x_ref[16]
    w_a = x_ref[17]
    h_a = x_ref[18]
    xc = dx * w_a + x_a
    yc = dy * h_a + y_a
    w = jnp.exp(dw) * w_a
    h = jnp.exp(dh) * h_a
    ymin = yc - h / 2
    xmin = xc - w / 2
    ymax = yc + h / 2
    xmax = xc + w / 2
    chans_ref[0] = ymin
    chans_ref[1] = xmin
    chans_ref[2] = ymax
    chans_ref[3] = xmax
    for j in range(5):
        chans_ref[4 + 2 * j] = (x_ref[5 + 2 * j] * f32(V0)) * w_a + x_a
        chans_ref[5 + 2 * j] = (x_ref[6 + 2 * j] * f32(V0)) * h_a + y_a
    chans_ref[14] = (ymax - ymin) * (xmax - xmin)
    s_ref[...] = jnp.where(sc >= f32(NMS_THRESH), sc, NEG_INF)

    gid = (lax.broadcasted_iota(jnp.int32, (ROWS, LANES), 0) * LANES
           + lax.broadcasted_iota(jnp.int32, (ROWS, LANES), 1))
    lane_iota = lax.broadcasted_iota(jnp.int32, (1, LANES), 1)
    out_iota = lax.broadcasted_iota(jnp.int32, (1, 16), 1)

    def body(i, carry):
        s = s_ref[...]
        maxv = jnp.max(s, axis=(0, 1), keepdims=True)
        minv = jnp.min(jnp.where(s == maxv, gid, INT_MAX),
                       axis=(0, 1), keepdims=True)
        pos2d = gid == minv
        vals = [jnp.sum(jnp.where(pos2d, chans_ref[c], f32(0.0)),
                        axis=(0, 1), keepdims=True) for c in range(4)]
        sy0, sx0, sy1, sx1 = vals
        area1 = (sy1 - sy0) * (sx1 - sx0)
        iy0 = jnp.maximum(sy0, chans_ref[0])
        ix0 = jnp.maximum(sx0, chans_ref[1])
        iy1 = jnp.minimum(sy1, chans_ref[2])
        ix1 = jnp.minimum(sx1, chans_ref[3])
        inter = (jnp.maximum(iy1 - iy0, f32(0.0))
                 * jnp.maximum(ix1 - ix0, f32(0.0)))
        iou = inter / (area1 + chans_ref[14] - inter + f32(1e-8))
        kill = (iou > f32(NMS_THRESH)) | pos2d
        s_ref[...] = jnp.where(kill, NEG_INF, s)
        okf = jnp.where(maxv > NEG_INF, f32(1.0), f32(0.0))
        idx = minv[0, 0]
        r = idx // LANES
        lonehot = lane_iota == (idx - r * LANES)
        row = jnp.zeros((1, 16), jnp.float32)
        for c, v in enumerate(vals):
            row = jnp.where(out_iota == c, v, row)
        for c in range(4, 14):
            rv = chans_ref[c, pl.ds(r, 1), :]
            v = jnp.sum(jnp.where(lonehot, rv, f32(0.0)))
            row = jnp.where(out_iota == c, v, row)
        out_ref[pl.ds(i, 1), :] = row * okf
        return carry

    lax.fori_loop(0, MAX_OUT, body, 0, unroll=2)


def kernel(cls_pred, reg_pred, lnd_pred, anchors):
    scores = cls_pred[0, :, 1]
    x = jnp.concatenate(
        [scores[:, None], reg_pred[0], lnd_pred[0], anchors], axis=1)
    xt = jnp.pad(x.T, ((0, 0), (0, NPAD - N))).reshape(19, ROWS, LANES)
    out = pl.pallas_call(
        _nms_body,
        out_shape=jax.ShapeDtypeStruct((MAX_OUT, 16), jnp.float32),
        scratch_shapes=[
            pltpu.VMEM((NCH, ROWS, LANES), jnp.float32),
            pltpu.VMEM((ROWS, LANES), jnp.float32),
        ],
    )(xt)
    return out[:, :4], out[:, 4:14]


# speedup vs baseline: 1.3355x; 1.0502x over previous
import jax
import jax.numpy as jnp
from jax import lax
from jax.experimental import pallas as pl
from jax.experimental.pallas import tpu as pltpu

N = 20000
LANES = 128
ROWS = (N + LANES - 1) // LANES
ROWS = ((ROWS + 7) // 8) * 8
NPAD = ROWS * LANES
MAX_OUT = 200
NMS_THRESH = 0.4
V0 = 0.1
V1 = 0.2
NEG_INF = float("-inf")
INT_MAX = 2**31 - 1

NCH = 15


def _nms_body(x_ref, out_ref, chans_ref):
    f32 = jnp.float32
    sc = x_ref[0]
    dx = x_ref[1] * f32(V0)
    dy = x_ref[2] * f32(V0)
    dw = x_ref[3] * f32(V1)
    dh = x_ref[4] * f32(V1)
    x_a = x_ref[15]
    y_a = x_ref[16]
    w_a = x_ref[17]
    h_a = x_ref[18]
    xc = dx * w_a + x_a
    yc = dy * h_a + y_a
    w = jnp.exp(dw) * w_a
    h = jnp.exp(dh) * h_a
    ymin = yc - h / 2
    xmin = xc - w / 2
    ymax = yc + h / 2
    xmax = xc + w / 2
    chans_ref[0] = ymin
    chans_ref[1] = xmin
    chans_ref[2] = ymax
    chans_ref[3] = xmax
    for j in range(5):
        chans_ref[4 + 2 * j] = (x_ref[5 + 2 * j] * f32(V0)) * w_a + x_a
        chans_ref[5 + 2 * j] = (x_ref[6 + 2 * j] * f32(V0)) * h_a + y_a
    chans_ref[14] = (ymax - ymin) * (xmax - xmin)

    gid = (lax.broadcasted_iota(jnp.int32, (ROWS, LANES), 0) * LANES
           + lax.broadcasted_iota(jnp.int32, (ROWS, LANES), 1))
    lane_iota = lax.broadcasted_iota(jnp.int32, (1, LANES), 1)
    out_iota = lax.broadcasted_iota(jnp.int32, (1, 16), 1)

    def body(i, s):
        m = jnp.max(s)
        ok = m > NEG_INF
        idx = jnp.min(jnp.where(s == m, gid, INT_MAX))
        r = idx // LANES
        lane = idx - r * LANES
        lonehot = lane_iota == lane
        vals = []
        for c in range(14):
            rv = chans_ref[c, pl.ds(r, 1), :]
            vals.append(jnp.sum(jnp.where(lonehot, rv, f32(0.0))))
        sy0, sx0, sy1, sx1 = vals[0], vals[1], vals[2], vals[3]
        area1 = (sy1 - sy0) * (sx1 - sx0)
        iy0 = jnp.maximum(sy0, chans_ref[0])
        ix0 = jnp.maximum(sx0, chans_ref[1])
        iy1 = jnp.minimum(sy1, chans_ref[2])
        ix1 = jnp.minimum(sx1, chans_ref[3])
        inter = (jnp.maximum(iy1 - iy0, f32(0.0))
                 * jnp.maximum(ix1 - ix0, f32(0.0)))
        iou = inter / (area1 + chans_ref[14] - inter + f32(1e-8))
        kill = (iou > f32(NMS_THRESH)) | (gid == idx)
        s_new = jnp.where(kill, NEG_INF, s)
        okf = jnp.where(ok, f32(1.0), f32(0.0))
        row = jnp.zeros((1, 16), jnp.float32)
        for c, v in enumerate(vals):
            row = jnp.where(out_iota == c, v, row)
        out_ref[pl.ds(i, 1), :] = row * okf
        return s_new

    s0 = jnp.where(sc >= f32(NMS_THRESH), sc, NEG_INF)
    lax.fori_loop(0, MAX_OUT, body, s0, unroll=2)


def kernel(cls_pred, reg_pred, lnd_pred, anchors):
    scores = cls_pred[0, :, 1]
    x = jnp.concatenate(
        [scores[:, None], reg_pred[0], lnd_pred[0], anchors], axis=1)
    xt = jnp.pad(x.T, ((0, 0), (0, NPAD - N))).reshape(19, ROWS, LANES)
    out = pl.pallas_call(
        _nms_body,
        out_shape=jax.ShapeDtypeStruct((MAX_OUT, 16), jnp.float32),
        scratch_shapes=[
            pltpu.VMEM((NCH, ROWS, LANES), jnp.float32),
        ],
    )(xt)
    return out[:, :4], out[:, 4:14]
